# baseline (speedup 1.0000x reference)
import functools

import jax
import jax.numpy as jnp
from jax import lax
from jax.experimental import pallas as pl
from jax.experimental.pallas import tpu as pltpu
from jax.experimental.pallas import tpu_sc as plsc

N_NODES = 100000
D_FEAT = 128
NUM_GRAPHS = 512
HIDDEN = 256

NC = 2
NS = 16
NW = NC * NS
ROWS_W = 3136
CHUNK = 192
NPAIRS = 9
NGROUPS = CHUNK // 16
NSLICE = D_FEAT // 16

_GDN = lax.GatherDimensionNumbers(
    offset_dims=(), collapsed_slice_dims=(0,), start_index_map=(0,))


def _lane_bcast(v, r):
    idx = jnp.full((16,), r, dtype=jnp.int32)
    return lax.gather(v, idx[:, None], _GDN, (1,),
                      mode=lax.GatherScatterMode.PROMISE_IN_BOUNDS)


def _sc_body(feat_hbm, ids_hbm, out_hbm, cnt_hbm,
             rowA, idsA, rowB, idsB, acc, cnt, semA, semB):
    cid = lax.axis_index("c")
    sid = lax.axis_index("s")
    wid = sid * NC + cid
    base = wid * ROWS_W
    end = jnp.minimum(base + ROWS_W, N_NODES)

    zero16 = jnp.zeros((16,), jnp.float32)
    ones16 = jnp.ones((16,), jnp.float32)
    iota16 = lax.iota(jnp.int32, 16)

    def start_chunk(k, rowbuf, idsbuf, sem):
        b = jnp.minimum(base + k * CHUNK, end - CHUNK)
        pltpu.async_copy(feat_hbm.at[pl.ds(b, CHUNK), :], rowbuf, sem)
        pltpu.async_copy(ids_hbm.at[pl.ds(b, CHUNK)], idsbuf, sem)

    def wait_chunk(rowbuf, idsbuf, sem):
        pltpu.make_async_copy(
            feat_hbm.at[pl.ds(0, CHUNK), :], rowbuf, sem).wait()
        pltpu.make_async_copy(
            ids_hbm.at[pl.ds(0, CHUNK)], idsbuf, sem).wait()

    start_chunk(0, rowA, idsA, semA)

    def zrow(i, carry):
        for j in range(NSLICE):
            acc[i, pl.ds(j * 16, 16)] = zero16
        return carry

    lax.fori_loop(0, NUM_GRAPHS, zrow, 0)

    def zcnt(c, carry):
        for r in range(16):
            cnt[r, pl.ds(c * 16, 16)] = zero16
        return carry

    lax.fori_loop(0, NUM_GRAPHS // 16, zcnt, 0)

    def make_group_body(rowbuf, idsbuf):
        def group_body(g, carry):
            idv = idsbuf[pl.ds(g * 16, 16)]
            plsc.addupdate_scatter(cnt, [iota16, idv], ones16)
            prev = None
            for r in range(16):
                seg = _lane_bcast(idv, r)
                row = g * 16 + r
                xs = [rowbuf[row, pl.ds(j * 16, 16)] for j in range(NSLICE)]
                if prev is not None:
                    pseg, pxs = prev
                    for j in range(NSLICE):
                        plsc.addupdate_scatter(
                            acc, [pseg, iota16 + j * 16], pxs[j])
                prev = (seg, xs)
            pseg, pxs = prev
            for j in range(NSLICE):
                plsc.addupdate_scatter(acc, [pseg, iota16 + j * 16], pxs[j])
            return carry
        return group_body

    def process(k, p, rowbuf, idsbuf):
        b = jnp.minimum(base + k * CHUNK, end - CHUNK)
        gs = (p - b) // 16
        lax.fori_loop(gs, NGROUPS, make_group_body(rowbuf, idsbuf), 0)
        return b + CHUNK

    def pair_body(t, p):
        k0 = 2 * t
        start_chunk(k0 + 1, rowB, idsB, semB)
        wait_chunk(rowA, idsA, semA)
        p = process(k0, p, rowA, idsA)

        @pl.when(t < NPAIRS - 1)
        def _():
            start_chunk(k0 + 2, rowA, idsA, semA)

        wait_chunk(rowB, idsB, semB)
        return process(k0 + 1, p, rowB, idsB)

    lax.fori_loop(0, NPAIRS, pair_body, base)

    pltpu.sync_copy(acc, out_hbm.at[wid])
    pltpu.sync_copy(cnt, cnt_hbm.at[wid])


_sc_seg_sum = functools.partial(
    pl.kernel,
    out_type=[
        jax.ShapeDtypeStruct((NW, NUM_GRAPHS, D_FEAT), jnp.float32),
        jax.ShapeDtypeStruct((NW, 16, NUM_GRAPHS), jnp.float32),
    ],
    mesh=plsc.VectorSubcoreMesh(
        core_axis_name="c", subcore_axis_name="s",
        num_cores=NC, num_subcores=NS),
    compiler_params=pltpu.CompilerParams(
        needs_layout_passes=False, use_tc_tiling_on_sc=False),
    scratch_types=[
        pltpu.VMEM((CHUNK, D_FEAT), jnp.float32),
        pltpu.VMEM((CHUNK,), jnp.int32),
        pltpu.VMEM((CHUNK, D_FEAT), jnp.float32),
        pltpu.VMEM((CHUNK,), jnp.int32),
        pltpu.VMEM((NUM_GRAPHS, D_FEAT), jnp.float32),
        pltpu.VMEM((16, NUM_GRAPHS), jnp.float32),
        pltpu.SemaphoreType.DMA,
        pltpu.SemaphoreType.DMA,
    ],
)(_sc_body)


def _tc_body(p_ref, c_ref, W1_ref, b1_ref, W2_ref, b2_ref, W3_ref, b3_ref,
             out_ref, acc_ref, cacc_ref):
    i = pl.program_id(0)

    @pl.when(i == 0)
    def _init():
        acc_ref[...] = p_ref[0]
        cacc_ref[...] = c_ref[0]

    @pl.when(i > 0)
    def _accum():
        acc_ref[...] += p_ref[0]
        cacc_ref[...] += c_ref[0]

    @pl.when(i == NW - 1)
    def _finish():
        c_col = jax.lax.dot_general(
            cacc_ref[...], jnp.ones((16, 1), jnp.float32),
            (((0,), (0,)), ((), ())),
            preferred_element_type=jnp.float32)
        pooled = acc_ref[...] / jnp.maximum(c_col, 1.0)
        h = jnp.maximum(
            jnp.dot(pooled, W1_ref[...], preferred_element_type=jnp.float32)
            + b1_ref[...], 0.0)
        h = jnp.maximum(
            jnp.dot(h, W2_ref[...], preferred_element_type=jnp.float32)
            + b2_ref[...], 0.0)
        out_ref[...] = (
            jnp.dot(h, W3_ref[...], preferred_element_type=jnp.float32)
            + b3_ref[...])


def kernel(feat, segment_ids, W1, b1, W2, b2, W3, b3):
    ids = segment_ids.astype(jnp.int32)
    partials, counts = _sc_seg_sum(feat, ids)
    pred = pl.pallas_call(
        _tc_body,
        grid=(NW,),
        in_specs=[
            pl.BlockSpec((1, NUM_GRAPHS, D_FEAT), lambda i: (i, 0, 0)),
            pl.BlockSpec((1, 16, NUM_GRAPHS), lambda i: (i, 0, 0)),
            pl.BlockSpec((D_FEAT, HIDDEN), lambda i: (0, 0)),
            pl.BlockSpec((1, HIDDEN), lambda i: (0, 0)),
            pl.BlockSpec((HIDDEN, HIDDEN), lambda i: (0, 0)),
            pl.BlockSpec((1, HIDDEN), lambda i: (0, 0)),
            pl.BlockSpec((HIDDEN, 1), lambda i: (0, 0)),
            pl.BlockSpec((1, 1), lambda i: (0, 0)),
        ],
        out_specs=pl.BlockSpec((NUM_GRAPHS, 1), lambda i: (0, 0)),
        out_shape=jax.ShapeDtypeStruct((NUM_GRAPHS, 1), jnp.float32),
        scratch_shapes=[
            pltpu.VMEM((NUM_GRAPHS, D_FEAT), jnp.float32),
            pltpu.VMEM((16, NUM_GRAPHS), jnp.float32),
        ],
    )(partials, counts,
      W1, b1.reshape(1, HIDDEN), W2, b2.reshape(1, HIDDEN),
      W3, b3.reshape(1, 1))
    return pred.reshape(NUM_GRAPHS)


# speedup vs baseline: 1.1566x; 1.1566x over previous
import functools

import jax
import jax.numpy as jnp
from jax import lax
from jax.experimental import pallas as pl
from jax.experimental.pallas import tpu as pltpu
from jax.experimental.pallas import tpu_sc as plsc

N_NODES = 100000
D_FEAT = 128
NUM_GRAPHS = 512
HIDDEN = 256

NC = 2
NS = 16
NW = NC * NS
ROWS_W = 3136
CHUNK = 192
NPAIRS = 9
NGROUPS = CHUNK // 16
NSLICE = D_FEAT // 16
TCBLK = 4

_GDN = lax.GatherDimensionNumbers(
    offset_dims=(), collapsed_slice_dims=(0,), start_index_map=(0,))


def _lane_bcast(v, r):
    idx = jnp.full((16,), r, dtype=jnp.int32)
    return lax.gather(v, idx[:, None], _GDN, (1,),
                      mode=lax.GatherScatterMode.PROMISE_IN_BOUNDS)


def _sc_body(feat_hbm, ids_hbm, out_hbm, cnt_hbm,
             rowA, idsA, rowB, idsB, acc, cnt, semA, semB):
    cid = lax.axis_index("c")
    sid = lax.axis_index("s")
    wid = sid * NC + cid
    base = wid * ROWS_W
    end = jnp.minimum(base + ROWS_W, N_NODES)

    zero16 = jnp.zeros((16,), jnp.float32)
    ones16 = jnp.ones((16,), jnp.float32)
    iota16 = lax.iota(jnp.int32, 16)

    def start_chunk(k, rowbuf, idsbuf, sem):
        b = jnp.minimum(base + k * CHUNK, end - CHUNK)
        pltpu.async_copy(feat_hbm.at[pl.ds(b, CHUNK), :], rowbuf, sem)
        pltpu.async_copy(ids_hbm.at[pl.ds(b, CHUNK)], idsbuf, sem)

    def wait_chunk(rowbuf, idsbuf, sem):
        pltpu.make_async_copy(
            feat_hbm.at[pl.ds(0, CHUNK), :], rowbuf, sem).wait()
        pltpu.make_async_copy(
            ids_hbm.at[pl.ds(0, CHUNK)], idsbuf, sem).wait()

    start_chunk(0, rowA, idsA, semA)

    def zrow(i, carry):
        for j in range(NSLICE):
            acc[i, pl.ds(j * 16, 16)] = zero16
        return carry

    lax.fori_loop(0, NUM_GRAPHS, zrow, 0)

    def zcnt(c, carry):
        for r in range(16):
            cnt[r, pl.ds(c * 16, 16)] = zero16
        return carry

    lax.fori_loop(0, NUM_GRAPHS // 16, zcnt, 0)

    def make_group_body(rowbuf, idsbuf):
        def group_body(g, carry):
            idv = idsbuf[pl.ds(g * 16, 16)]
            plsc.addupdate_scatter(cnt, [iota16, idv], ones16)
            prev = None
            for r in range(16):
                seg = _lane_bcast(idv, r)
                row = g * 16 + r
                xs = [rowbuf[row, pl.ds(j * 16, 16)] for j in range(NSLICE)]
                if prev is not None:
                    pseg, pxs = prev
                    for j in range(NSLICE):
                        plsc.addupdate_scatter(
                            acc, [pseg, iota16 + j * 16], pxs[j])
                prev = (seg, xs)
            pseg, pxs = prev
            for j in range(NSLICE):
                plsc.addupdate_scatter(acc, [pseg, iota16 + j * 16], pxs[j])
            return carry
        return group_body

    def process(k, p, rowbuf, idsbuf):
        b = jnp.minimum(base + k * CHUNK, end - CHUNK)
        gs = (p - b) // 16
        lax.fori_loop(gs, NGROUPS, make_group_body(rowbuf, idsbuf), 0)
        return b + CHUNK

    def pair_body(t, p):
        k0 = 2 * t
        start_chunk(k0 + 1, rowB, idsB, semB)
        wait_chunk(rowA, idsA, semA)
        p = process(k0, p, rowA, idsA)

        @pl.when(t < NPAIRS - 1)
        def _():
            start_chunk(k0 + 2, rowA, idsA, semA)

        wait_chunk(rowB, idsB, semB)
        return process(k0 + 1, p, rowB, idsB)

    lax.fori_loop(0, NPAIRS, pair_body, base)

    pltpu.sync_copy(acc, out_hbm.at[wid])
    pltpu.sync_copy(cnt, cnt_hbm.at[wid])


_sc_seg_sum = functools.partial(
    pl.kernel,
    out_type=[
        jax.ShapeDtypeStruct((NW, NUM_GRAPHS, D_FEAT), jnp.float32),
        jax.ShapeDtypeStruct((NW, 16, NUM_GRAPHS), jnp.float32),
    ],
    mesh=plsc.VectorSubcoreMesh(
        core_axis_name="c", subcore_axis_name="s",
        num_cores=NC, num_subcores=NS),
    compiler_params=pltpu.CompilerParams(
        needs_layout_passes=False, use_tc_tiling_on_sc=False),
    scratch_types=[
        pltpu.VMEM((CHUNK, D_FEAT), jnp.float32),
        pltpu.VMEM((CHUNK,), jnp.int32),
        pltpu.VMEM((CHUNK, D_FEAT), jnp.float32),
        pltpu.VMEM((CHUNK,), jnp.int32),
        pltpu.VMEM((NUM_GRAPHS, D_FEAT), jnp.float32),
        pltpu.VMEM((16, NUM_GRAPHS), jnp.float32),
        pltpu.SemaphoreType.DMA,
        pltpu.SemaphoreType.DMA,
    ],
)(_sc_body)


def _tc_body(p_ref, c_ref, W1_ref, b1_ref, W2_ref, b2_ref, W3_ref, b3_ref,
             out_ref, acc_ref, cacc_ref):
    i = pl.program_id(0)
    psum = p_ref[0]
    csum = c_ref[0]
    for w in range(1, TCBLK):
        psum = psum + p_ref[w]
        csum = csum + c_ref[w]

    @pl.when(i == 0)
    def _init():
        acc_ref[...] = psum
        cacc_ref[...] = csum

    @pl.when(i > 0)
    def _accum():
        acc_ref[...] += psum
        cacc_ref[...] += csum

    @pl.when(i == NW // TCBLK - 1)
    def _finish():
        c_col = jax.lax.dot_general(
            cacc_ref[...], jnp.ones((16, 1), jnp.float32),
            (((0,), (0,)), ((), ())),
            preferred_element_type=jnp.float32)
        pooled = acc_ref[...] / jnp.maximum(c_col, 1.0)
        h = jnp.maximum(
            jnp.dot(pooled, W1_ref[...], preferred_element_type=jnp.float32)
            + b1_ref[...], 0.0)
        h = jnp.maximum(
            jnp.dot(h, W2_ref[...], preferred_element_type=jnp.float32)
            + b2_ref[...], 0.0)
        out_ref[...] = (
            jnp.dot(h, W3_ref[...], preferred_element_type=jnp.float32)
            + b3_ref[...])


def kernel(feat, segment_ids, W1, b1, W2, b2, W3, b3):
    ids = segment_ids.astype(jnp.int32)
    partials, counts = _sc_seg_sum(feat, ids)
    pred = pl.pallas_call(
        _tc_body,
        grid=(NW // TCBLK,),
        in_specs=[
            pl.BlockSpec((TCBLK, NUM_GRAPHS, D_FEAT), lambda i: (i, 0, 0)),
            pl.BlockSpec((TCBLK, 16, NUM_GRAPHS), lambda i: (i, 0, 0)),
            pl.BlockSpec((D_FEAT, HIDDEN), lambda i: (0, 0)),
            pl.BlockSpec((1, HIDDEN), lambda i: (0, 0)),
            pl.BlockSpec((HIDDEN, HIDDEN), lambda i: (0, 0)),
            pl.BlockSpec((1, HIDDEN), lambda i: (0, 0)),
            pl.BlockSpec((HIDDEN, 1), lambda i: (0, 0)),
            pl.BlockSpec((1, 1), lambda i: (0, 0)),
        ],
        out_specs=pl.BlockSpec((NUM_GRAPHS, 1), lambda i: (0, 0)),
        out_shape=jax.ShapeDtypeStruct((NUM_GRAPHS, 1), jnp.float32),
        scratch_shapes=[
            pltpu.VMEM((NUM_GRAPHS, D_FEAT), jnp.float32),
            pltpu.VMEM((16, NUM_GRAPHS), jnp.float32),
        ],
    )(partials, counts,
      W1, b1.reshape(1, HIDDEN), W2, b2.reshape(1, HIDDEN),
      W3, b3.reshape(1, 1))
    return pred.reshape(NUM_GRAPHS)


# speedup vs baseline: 1.2327x; 1.0658x over previous
import functools

import jax
import jax.numpy as jnp
from jax import lax
from jax.experimental import pallas as pl
from jax.experimental.pallas import tpu as pltpu
from jax.experimental.pallas import tpu_sc as plsc

N_NODES = 100000
D_FEAT = 128
NUM_GRAPHS = 512
HIDDEN = 256

NC = 2
NS = 16
NW = NC * NS
ROWS_W = 3136
CHUNK = 192
NPAIRS = 9
NGROUPS = CHUNK // 16
NSLICE = D_FEAT // 16
TCBLK = 4

_GDN = lax.GatherDimensionNumbers(
    offset_dims=(), collapsed_slice_dims=(0,), start_index_map=(0,))


def _lane_bcast(v, r):
    idx = jnp.full((16,), r, dtype=jnp.int32)
    return lax.gather(v, idx[:, None], _GDN, (1,),
                      mode=lax.GatherScatterMode.PROMISE_IN_BOUNDS)


def _sc_body(feat_hbm, ids_hbm, out_hbm, cnt_hbm,
             rowA, idsA, rowB, idsB, acc, cnt, semA, semB):
    cid = lax.axis_index("c")
    sid = lax.axis_index("s")
    wid = sid * NC + cid
    base = wid * ROWS_W
    end = jnp.minimum(base + ROWS_W, N_NODES)

    zero16 = jnp.zeros((16,), jnp.float32)
    ones16 = jnp.ones((16,), jnp.float32)
    iota16 = lax.iota(jnp.int32, 16)

    def start_chunk(k, rowbuf, idsbuf, sem):
        b = jnp.minimum(base + k * CHUNK, end - CHUNK)
        pltpu.async_copy(feat_hbm.at[pl.ds(b, CHUNK), :], rowbuf, sem)
        pltpu.async_copy(ids_hbm.at[pl.ds(b, CHUNK)], idsbuf, sem)

    def wait_chunk(rowbuf, idsbuf, sem):
        pltpu.make_async_copy(
            feat_hbm.at[pl.ds(0, CHUNK), :], rowbuf, sem).wait()
        pltpu.make_async_copy(
            ids_hbm.at[pl.ds(0, CHUNK)], idsbuf, sem).wait()

    start_chunk(0, rowA, idsA, semA)

    def zrow(i, carry):
        for j in range(NSLICE):
            acc[i, pl.ds(j * 16, 16)] = zero16
        return carry

    lax.fori_loop(0, NUM_GRAPHS, zrow, 0)

    def zcnt(c, carry):
        for r in range(16):
            cnt[r, pl.ds(c * 16, 16)] = zero16
        return carry

    lax.fori_loop(0, NUM_GRAPHS // 16, zcnt, 0)

    def make_group_body(rowbuf, idsbuf):
        def group_body(g, carry):
            accs, segv = carry
            idv = idsbuf[pl.ds(g * 16, 16)]
            plsc.addupdate_scatter(cnt, [iota16, idv], ones16)
            nb = plsc.all_reduce_population_count(idv != segv)[0]

            def fast(ops):
                a, sv = ops
                new = list(a)
                for r in range(16):
                    row = g * 16 + r
                    for j in range(NSLICE):
                        new[j] = new[j] + rowbuf[row, pl.ds(j * 16, 16)]
                return tuple(new), sv

            def slow(ops):
                a, sv = ops
                tgt = jnp.maximum(sv, 0)
                for j in range(NSLICE):
                    plsc.addupdate_scatter(acc, [tgt, iota16 + j * 16], a[j])
                prev = None
                for r in range(16):
                    seg = _lane_bcast(idv, r)
                    row = g * 16 + r
                    xs = [rowbuf[row, pl.ds(j * 16, 16)]
                          for j in range(NSLICE)]
                    if prev is not None:
                        pseg, pxs = prev
                        for j in range(NSLICE):
                            plsc.addupdate_scatter(
                                acc, [pseg, iota16 + j * 16], pxs[j])
                    prev = (seg, xs)
                pseg, pxs = prev
                for j in range(NSLICE):
                    plsc.addupdate_scatter(acc, [pseg, iota16 + j * 16],
                                           pxs[j])
                zeros = tuple(zero16 for _ in range(NSLICE))
                return zeros, _lane_bcast(idv, 15)

            return lax.cond(nb == 0, fast, slow, (accs, segv))
        return group_body

    def process(k, state, rowbuf, idsbuf):
        p, accs, segv = state
        b = jnp.minimum(base + k * CHUNK, end - CHUNK)
        gs = (p - b) // 16
        accs, segv = lax.fori_loop(
            gs, NGROUPS, make_group_body(rowbuf, idsbuf), (accs, segv))
        return b + CHUNK, accs, segv

    def pair_body(t, state):
        k0 = 2 * t
        start_chunk(k0 + 1, rowB, idsB, semB)
        wait_chunk(rowA, idsA, semA)
        state = process(k0, state, rowA, idsA)

        @pl.when(t < NPAIRS - 1)
        def _():
            start_chunk(k0 + 2, rowA, idsA, semA)

        wait_chunk(rowB, idsB, semB)
        return process(k0 + 1, state, rowB, idsB)

    init_state = (base,
                  tuple(zero16 for _ in range(NSLICE)),
                  jnp.full((16,), -1, jnp.int32))
    _, accs, segv = lax.fori_loop(0, NPAIRS, pair_body, init_state)

    tgt = jnp.maximum(segv, 0)
    for j in range(NSLICE):
        plsc.addupdate_scatter(acc, [tgt, iota16 + j * 16], accs[j])

    pltpu.sync_copy(acc, out_hbm.at[wid])
    pltpu.sync_copy(cnt, cnt_hbm.at[wid])


_sc_seg_sum = functools.partial(
    pl.kernel,
    out_type=[
        jax.ShapeDtypeStruct((NW, NUM_GRAPHS, D_FEAT), jnp.float32),
        jax.ShapeDtypeStruct((NW, 16, NUM_GRAPHS), jnp.float32),
    ],
    mesh=plsc.VectorSubcoreMesh(
        core_axis_name="c", subcore_axis_name="s",
        num_cores=NC, num_subcores=NS),
    compiler_params=pltpu.CompilerParams(
        needs_layout_passes=False, use_tc_tiling_on_sc=False),
    scratch_types=[
        pltpu.VMEM((CHUNK, D_FEAT), jnp.float32),
        pltpu.VMEM((CHUNK,), jnp.int32),
        pltpu.VMEM((CHUNK, D_FEAT), jnp.float32),
        pltpu.VMEM((CHUNK,), jnp.int32),
        pltpu.VMEM((NUM_GRAPHS, D_FEAT), jnp.float32),
        pltpu.VMEM((16, NUM_GRAPHS), jnp.float32),
        pltpu.SemaphoreType.DMA,
        pltpu.SemaphoreType.DMA,
    ],
)(_sc_body)


def _tc_body(p_ref, c_ref, W1_ref, b1_ref, W2_ref, b2_ref, W3_ref, b3_ref,
             out_ref, acc_ref, cacc_ref):
    i = pl.program_id(0)
    psum = p_ref[0]
    csum = c_ref[0]
    for w in range(1, TCBLK):
        psum = psum + p_ref[w]
        csum = csum + c_ref[w]

    @pl.when(i == 0)
    def _init():
        acc_ref[...] = psum
        cacc_ref[...] = csum

    @pl.when(i > 0)
    def _accum():
        acc_ref[...] += psum
        cacc_ref[...] += csum

    @pl.when(i == NW // TCBLK - 1)
    def _finish():
        c_col = jax.lax.dot_general(
            cacc_ref[...], jnp.ones((16, 1), jnp.float32),
            (((0,), (0,)), ((), ())),
            preferred_element_type=jnp.float32)
        pooled = acc_ref[...] / jnp.maximum(c_col, 1.0)
        h = jnp.maximum(
            jnp.dot(pooled, W1_ref[...], preferred_element_type=jnp.float32)
            + b1_ref[...], 0.0)
        h = jnp.maximum(
            jnp.dot(h, W2_ref[...], preferred_element_type=jnp.float32)
            + b2_ref[...], 0.0)
        out_ref[...] = (
            jnp.dot(h, W3_ref[...], preferred_element_type=jnp.float32)
            + b3_ref[...])


def kernel(feat, segment_ids, W1, b1, W2, b2, W3, b3):
    ids = segment_ids.astype(jnp.int32)
    partials, counts = _sc_seg_sum(feat, ids)
    pred = pl.pallas_call(
        _tc_body,
        grid=(NW // TCBLK,),
        in_specs=[
            pl.BlockSpec((TCBLK, NUM_GRAPHS, D_FEAT), lambda i: (i, 0, 0)),
            pl.BlockSpec((TCBLK, 16, NUM_GRAPHS), lambda i: (i, 0, 0)),
            pl.BlockSpec((D_FEAT, HIDDEN), lambda i: (0, 0)),
            pl.BlockSpec((1, HIDDEN), lambda i: (0, 0)),
            pl.BlockSpec((HIDDEN, HIDDEN), lambda i: (0, 0)),
            pl.BlockSpec((1, HIDDEN), lambda i: (0, 0)),
            pl.BlockSpec((HIDDEN, 1), lambda i: (0, 0)),
            pl.BlockSpec((1, 1), lambda i: (0, 0)),
        ],
        out_specs=pl.BlockSpec((NUM_GRAPHS, 1), lambda i: (0, 0)),
        out_shape=jax.ShapeDtypeStruct((NUM_GRAPHS, 1), jnp.float32),
        scratch_shapes=[
            pltpu.VMEM((NUM_GRAPHS, D_FEAT), jnp.float32),
            pltpu.VMEM((16, NUM_GRAPHS), jnp.float32),
        ],
    )(partials, counts,
      W1, b1.reshape(1, HIDDEN), W2, b2.reshape(1, HIDDEN),
      W3, b3.reshape(1, 1))
    return pred.reshape(NUM_GRAPHS)


# speedup vs baseline: 1.2630x; 1.0246x over previous
import functools

import jax
import jax.numpy as jnp
from jax import lax
from jax.experimental import pallas as pl
from jax.experimental.pallas import tpu as pltpu
from jax.experimental.pallas import tpu_sc as plsc

N_NODES = 100000
D_FEAT = 128
NUM_GRAPHS = 512
HIDDEN = 256

NC = 2
NS = 16
NW = NC * NS
ROWS_W = 3136
CHUNK = 208
NPAIRS = 8
NGROUPS = CHUNK // 16
NSLICE = D_FEAT // 16
TCBLK = 4

_GDN = lax.GatherDimensionNumbers(
    offset_dims=(), collapsed_slice_dims=(0,), start_index_map=(0,))


def _lane_bcast(v, r):
    idx = jnp.full((16,), r, dtype=jnp.int32)
    return lax.gather(v, idx[:, None], _GDN, (1,),
                      mode=lax.GatherScatterMode.PROMISE_IN_BOUNDS)


def _sc_body(feat_hbm, ids_hbm, out_hbm, cnt_hbm,
             rowA, idsA, rowB, idsB, acc, cnt, semA, semB):
    cid = lax.axis_index("c")
    sid = lax.axis_index("s")
    wid = sid * NC + cid
    base = wid * ROWS_W
    end = jnp.minimum(base + ROWS_W, N_NODES)

    zero16 = jnp.zeros((16,), jnp.float32)
    ones16 = jnp.ones((16,), jnp.float32)
    iota16 = lax.iota(jnp.int32, 16)

    def start_chunk(k, rowbuf, idsbuf, sem):
        b = jnp.minimum(base + k * CHUNK, end - CHUNK)
        pltpu.async_copy(feat_hbm.at[pl.ds(b, CHUNK), :], rowbuf, sem)
        pltpu.async_copy(ids_hbm.at[pl.ds(b, CHUNK)], idsbuf, sem)

    def wait_chunk(rowbuf, idsbuf, sem):
        pltpu.make_async_copy(
            feat_hbm.at[pl.ds(0, CHUNK), :], rowbuf, sem).wait()
        pltpu.make_async_copy(
            ids_hbm.at[pl.ds(0, CHUNK)], idsbuf, sem).wait()

    start_chunk(0, rowA, idsA, semA)

    def zrow(i, carry):
        for j in range(NSLICE):
            acc[i, pl.ds(j * 16, 16)] = zero16
        return carry

    lax.fori_loop(0, NUM_GRAPHS, zrow, 0)

    def zcnt(c, carry):
        for r in range(16):
            cnt[r, pl.ds(c * 16, 16)] = zero16
        return carry

    lax.fori_loop(0, NUM_GRAPHS // 16, zcnt, 0)

    def make_group_body(rowbuf, idsbuf):
        def group_body(g, carry):
            accs, segv = carry
            idv = idsbuf[pl.ds(g * 16, 16)]
            plsc.addupdate_scatter(cnt, [iota16, idv], ones16)
            nb = plsc.all_reduce_population_count(idv != segv)[0]

            def fast(ops):
                a, sv = ops
                new = list(a)
                for r in range(16):
                    row = g * 16 + r
                    for j in range(NSLICE):
                        new[j] = new[j] + rowbuf[row, pl.ds(j * 16, 16)]
                return tuple(new), sv

            def slow(ops):
                a, sv = ops
                tgt = jnp.maximum(sv, 0)
                for j in range(NSLICE):
                    plsc.addupdate_scatter(acc, [tgt, iota16 + j * 16], a[j])
                prev = None
                for r in range(16):
                    seg = _lane_bcast(idv, r)
                    row = g * 16 + r
                    xs = [rowbuf[row, pl.ds(j * 16, 16)]
                          for j in range(NSLICE)]
                    if prev is not None:
                        pseg, pxs = prev
                        for j in range(NSLICE):
                            plsc.addupdate_scatter(
                                acc, [pseg, iota16 + j * 16], pxs[j])
                    prev = (seg, xs)
                pseg, pxs = prev
                for j in range(NSLICE):
                    plsc.addupdate_scatter(acc, [pseg, iota16 + j * 16],
                                           pxs[j])
                zeros = tuple(zero16 for _ in range(NSLICE))
                return zeros, _lane_bcast(idv, 15)

            return lax.cond(nb == 0, fast, slow, (accs, segv))
        return group_body

    def process(k, state, rowbuf, idsbuf):
        p, accs, segv = state
        b = jnp.minimum(base + k * CHUNK, end - CHUNK)
        gs = (p - b) // 16
        accs, segv = lax.fori_loop(
            gs, NGROUPS, make_group_body(rowbuf, idsbuf), (accs, segv))
        return b + CHUNK, accs, segv

    def pair_body(t, state):
        k0 = 2 * t
        start_chunk(k0 + 1, rowB, idsB, semB)
        wait_chunk(rowA, idsA, semA)
        state = process(k0, state, rowA, idsA)

        @pl.when(t < NPAIRS - 1)
        def _():
            start_chunk(k0 + 2, rowA, idsA, semA)

        wait_chunk(rowB, idsB, semB)
        return process(k0 + 1, state, rowB, idsB)

    init_state = (base,
                  tuple(zero16 for _ in range(NSLICE)),
                  jnp.full((16,), -1, jnp.int32))
    _, accs, segv = lax.fori_loop(0, NPAIRS, pair_body, init_state)

    tgt = jnp.maximum(segv, 0)
    for j in range(NSLICE):
        plsc.addupdate_scatter(acc, [tgt, iota16 + j * 16], accs[j])

    pltpu.sync_copy(acc, out_hbm.at[wid])
    pltpu.sync_copy(cnt, cnt_hbm.at[wid])


_sc_seg_sum = functools.partial(
    pl.kernel,
    out_type=[
        jax.ShapeDtypeStruct((NW, NUM_GRAPHS, D_FEAT), jnp.float32),
        jax.ShapeDtypeStruct((NW, 16, NUM_GRAPHS), jnp.float32),
    ],
    mesh=plsc.VectorSubcoreMesh(
        core_axis_name="c", subcore_axis_name="s",
        num_cores=NC, num_subcores=NS),
    compiler_params=pltpu.CompilerParams(
        needs_layout_passes=False, use_tc_tiling_on_sc=False),
    scratch_types=[
        pltpu.VMEM((CHUNK, D_FEAT), jnp.float32),
        pltpu.VMEM((CHUNK,), jnp.int32),
        pltpu.VMEM((CHUNK, D_FEAT), jnp.float32),
        pltpu.VMEM((CHUNK,), jnp.int32),
        pltpu.VMEM((NUM_GRAPHS, D_FEAT), jnp.float32),
        pltpu.VMEM((16, NUM_GRAPHS), jnp.float32),
        pltpu.SemaphoreType.DMA,
        pltpu.SemaphoreType.DMA,
    ],
)(_sc_body)


def _tc_body(p_ref, c_ref, W1_ref, b1_ref, W2_ref, b2_ref, W3_ref, b3_ref,
             out_ref, acc_ref, cacc_ref):
    i = pl.program_id(0)
    psum = p_ref[0]
    csum = c_ref[0]
    for w in range(1, TCBLK):
        psum = psum + p_ref[w]
        csum = csum + c_ref[w]

    @pl.when(i == 0)
    def _init():
        acc_ref[...] = psum
        cacc_ref[...] = csum

    @pl.when(i > 0)
    def _accum():
        acc_ref[...] += psum
        cacc_ref[...] += csum

    @pl.when(i == NW // TCBLK - 1)
    def _finish():
        c_col = jax.lax.dot_general(
            cacc_ref[...], jnp.ones((16, 1), jnp.float32),
            (((0,), (0,)), ((), ())),
            preferred_element_type=jnp.float32)
        pooled = acc_ref[...] / jnp.maximum(c_col, 1.0)
        h = jnp.maximum(
            jnp.dot(pooled, W1_ref[...], preferred_element_type=jnp.float32)
            + b1_ref[...], 0.0)
        h = jnp.maximum(
            jnp.dot(h, W2_ref[...], preferred_element_type=jnp.float32)
            + b2_ref[...], 0.0)
        out_ref[...] = (
            jnp.dot(h, W3_ref[...], preferred_element_type=jnp.float32)
            + b3_ref[...])


def kernel(feat, segment_ids, W1, b1, W2, b2, W3, b3):
    ids = segment_ids.astype(jnp.int32)
    partials, counts = _sc_seg_sum(feat, ids)
    pred = pl.pallas_call(
        _tc_body,
        grid=(NW // TCBLK,),
        in_specs=[
            pl.BlockSpec((TCBLK, NUM_GRAPHS, D_FEAT), lambda i: (i, 0, 0)),
            pl.BlockSpec((TCBLK, 16, NUM_GRAPHS), lambda i: (i, 0, 0)),
            pl.BlockSpec((D_FEAT, HIDDEN), lambda i: (0, 0)),
            pl.BlockSpec((1, HIDDEN), lambda i: (0, 0)),
            pl.BlockSpec((HIDDEN, HIDDEN), lambda i: (0, 0)),
            pl.BlockSpec((1, HIDDEN), lambda i: (0, 0)),
            pl.BlockSpec((HIDDEN, 1), lambda i: (0, 0)),
            pl.BlockSpec((1, 1), lambda i: (0, 0)),
        ],
        out_specs=pl.BlockSpec((NUM_GRAPHS, 1), lambda i: (0, 0)),
        out_shape=jax.ShapeDtypeStruct((NUM_GRAPHS, 1), jnp.float32),
        scratch_shapes=[
            pltpu.VMEM((NUM_GRAPHS, D_FEAT), jnp.float32),
            pltpu.VMEM((16, NUM_GRAPHS), jnp.float32),
        ],
    )(partials, counts,
      W1, b1.reshape(1, HIDDEN), W2, b2.reshape(1, HIDDEN),
      W3, b3.reshape(1, 1))
    return pred.reshape(NUM_GRAPHS)


# speedup vs baseline: 1.2634x; 1.0004x over previous
import functools

import jax
import jax.numpy as jnp
from jax import lax
from jax.experimental import pallas as pl
from jax.experimental.pallas import tpu as pltpu
from jax.experimental.pallas import tpu_sc as plsc

N_NODES = 100000
D_FEAT = 128
NUM_GRAPHS = 512
HIDDEN = 256

NC = 2
NS = 16
NW = NC * NS
ROWS_W = 3136
CHUNK = 208
NPAIRS = 8
NGROUPS = CHUNK // 16
NSLICE = D_FEAT // 16
TCBLK = 4

_GDN = lax.GatherDimensionNumbers(
    offset_dims=(), collapsed_slice_dims=(0,), start_index_map=(0,))


def _lane_bcast(v, r):
    idx = jnp.full((16,), r, dtype=jnp.int32)
    return lax.gather(v, idx[:, None], _GDN, (1,),
                      mode=lax.GatherScatterMode.PROMISE_IN_BOUNDS)


def _sc_body(feat_hbm, ids_hbm, out_hbm, cnt_hbm,
             rowA, idsA, rowB, idsB, acc, cnt, semA, semB):
    cid = lax.axis_index("c")
    sid = lax.axis_index("s")
    wid = sid * NC + cid
    base = wid * ROWS_W
    end = jnp.minimum(base + ROWS_W, N_NODES)

    zero16 = jnp.zeros((16,), jnp.float32)
    ones16 = jnp.ones((16,), jnp.float32)
    iota16 = lax.iota(jnp.int32, 16)

    def start_chunk(k, rowbuf, idsbuf, sem):
        b = jnp.minimum(base + k * CHUNK, end - CHUNK)
        pltpu.async_copy(feat_hbm.at[pl.ds(b, CHUNK), :], rowbuf, sem)
        pltpu.async_copy(ids_hbm.at[pl.ds(b, CHUNK)], idsbuf, sem)

    def wait_chunk(rowbuf, idsbuf, sem):
        pltpu.make_async_copy(
            feat_hbm.at[pl.ds(0, CHUNK), :], rowbuf, sem).wait()
        pltpu.make_async_copy(
            ids_hbm.at[pl.ds(0, CHUNK)], idsbuf, sem).wait()

    start_chunk(0, rowA, idsA, semA)

    def zrow(i, carry):
        for j in range(NSLICE):
            acc[i, pl.ds(j * 16, 16)] = zero16
        return carry

    lax.fori_loop(0, NUM_GRAPHS, zrow, 0)

    def zcnt(c, carry):
        for r in range(16):
            cnt[r, pl.ds(c * 16, 16)] = zero16
        return carry

    lax.fori_loop(0, NUM_GRAPHS // 16, zcnt, 0)

    def make_group_body(rowbuf, idsbuf):
        def group_body(g, carry):
            accs, segv = carry
            idv = idsbuf[pl.ds(g * 16, 16)]
            plsc.addupdate_scatter(cnt, [iota16, idv], ones16)
            nb = plsc.all_reduce_population_count(idv != segv)[0]

            def fast(ops):
                a, sv = ops
                new = list(a)
                for r in range(16):
                    row = g * 16 + r
                    for j in range(NSLICE):
                        new[j] = new[j] + rowbuf[row, pl.ds(j * 16, 16)]
                return tuple(new), sv

            def slow(ops):
                a, sv = ops
                tgt = jnp.maximum(sv, 0)
                for j in range(NSLICE):
                    plsc.addupdate_scatter(acc, [tgt, iota16 + j * 16], a[j])
                prev = None
                for r in range(16):
                    seg = _lane_bcast(idv, r)
                    row = g * 16 + r
                    xs = [rowbuf[row, pl.ds(j * 16, 16)]
                          for j in range(NSLICE)]
                    if prev is not None:
                        pseg, pxs = prev
                        for j in range(NSLICE):
                            plsc.addupdate_scatter(
                                acc, [pseg, iota16 + j * 16], pxs[j])
                    prev = (seg, xs)
                pseg, pxs = prev
                for j in range(NSLICE):
                    plsc.addupdate_scatter(acc, [pseg, iota16 + j * 16],
                                           pxs[j])
                zeros = tuple(zero16 for _ in range(NSLICE))
                return zeros, _lane_bcast(idv, 15)

            return lax.cond(nb == 0, fast, slow, (accs, segv))
        return group_body

    def process(k, state, rowbuf, idsbuf):
        p, accs, segv = state
        b = jnp.minimum(base + k * CHUNK, end - CHUNK)
        gs = (p - b) // 16
        accs, segv = lax.fori_loop(
            gs, NGROUPS, make_group_body(rowbuf, idsbuf), (accs, segv))
        return b + CHUNK, accs, segv

    def pair_body(t, state):
        k0 = 2 * t
        start_chunk(k0 + 1, rowB, idsB, semB)
        wait_chunk(rowA, idsA, semA)
        state = process(k0, state, rowA, idsA)

        @pl.when(t < NPAIRS - 1)
        def _():
            start_chunk(k0 + 2, rowA, idsA, semA)

        wait_chunk(rowB, idsB, semB)
        return process(k0 + 1, state, rowB, idsB)

    init_state = (base,
                  tuple(zero16 for _ in range(NSLICE)),
                  jnp.full((16,), -1, jnp.int32))
    _, accs, segv = lax.fori_loop(0, NPAIRS, pair_body, init_state)

    tgt = jnp.maximum(segv, 0)
    for j in range(NSLICE):
        plsc.addupdate_scatter(acc, [tgt, iota16 + j * 16], accs[j])

    pltpu.sync_copy(acc, out_hbm.at[pl.ds(wid * NUM_GRAPHS, NUM_GRAPHS), :])
    pltpu.sync_copy(cnt, cnt_hbm.at[pl.ds(wid * 16, 16), :])


_sc_seg_sum = functools.partial(
    pl.kernel,
    out_type=[
        jax.ShapeDtypeStruct((NW * NUM_GRAPHS, D_FEAT), jnp.float32),
        jax.ShapeDtypeStruct((NW * 16, NUM_GRAPHS), jnp.float32),
    ],
    mesh=plsc.VectorSubcoreMesh(
        core_axis_name="c", subcore_axis_name="s",
        num_cores=NC, num_subcores=NS),
    compiler_params=pltpu.CompilerParams(
        needs_layout_passes=False, use_tc_tiling_on_sc=False),
    scratch_types=[
        pltpu.VMEM((CHUNK, D_FEAT), jnp.float32),
        pltpu.VMEM((CHUNK,), jnp.int32),
        pltpu.VMEM((CHUNK, D_FEAT), jnp.float32),
        pltpu.VMEM((CHUNK,), jnp.int32),
        pltpu.VMEM((NUM_GRAPHS, D_FEAT), jnp.float32),
        pltpu.VMEM((16, NUM_GRAPHS), jnp.float32),
        pltpu.SemaphoreType.DMA,
        pltpu.SemaphoreType.DMA,
    ],
)(_sc_body)


def _tc_body(p_ref, c_ref, W1_ref, b1_ref, W2_ref, b2_ref, W3_ref, b3_ref,
             out_ref, acc_ref, cacc_ref):
    i = pl.program_id(0)
    psum = p_ref[0:NUM_GRAPHS]
    csum = c_ref[0:16]
    for w in range(1, TCBLK):
        psum = psum + p_ref[w * NUM_GRAPHS:(w + 1) * NUM_GRAPHS]
        csum = csum + c_ref[w * 16:(w + 1) * 16]

    @pl.when(i == 0)
    def _init():
        acc_ref[...] = psum
        cacc_ref[...] = csum

    @pl.when(i > 0)
    def _accum():
        acc_ref[...] += psum
        cacc_ref[...] += csum

    @pl.when(i == NW // TCBLK - 1)
    def _finish():
        c_col = jax.lax.dot_general(
            cacc_ref[...], jnp.ones((16, 1), jnp.float32),
            (((0,), (0,)), ((), ())),
            preferred_element_type=jnp.float32)
        pooled = acc_ref[...] / jnp.maximum(c_col, 1.0)
        h = jnp.maximum(
            jnp.dot(pooled, W1_ref[...], preferred_element_type=jnp.float32)
            + b1_ref[...], 0.0)
        h = jnp.maximum(
            jnp.dot(h, W2_ref[...], preferred_element_type=jnp.float32)
            + b2_ref[...], 0.0)
        out_ref[...] = (
            jnp.dot(h, W3_ref[...], preferred_element_type=jnp.float32)
            + b3_ref[...])


def kernel(feat, segment_ids, W1, b1, W2, b2, W3, b3):
    ids = segment_ids.astype(jnp.int32)
    partials, counts = _sc_seg_sum(feat, ids)
    pred = pl.pallas_call(
        _tc_body,
        grid=(NW // TCBLK,),
        in_specs=[
            pl.BlockSpec((TCBLK * NUM_GRAPHS, D_FEAT), lambda i: (i, 0)),
            pl.BlockSpec((TCBLK * 16, NUM_GRAPHS), lambda i: (i, 0)),
            pl.BlockSpec((D_FEAT, HIDDEN), lambda i: (0, 0)),
            pl.BlockSpec((1, HIDDEN), lambda i: (0, 0)),
            pl.BlockSpec((HIDDEN, HIDDEN), lambda i: (0, 0)),
            pl.BlockSpec((1, HIDDEN), lambda i: (0, 0)),
            pl.BlockSpec((HIDDEN, 1), lambda i: (0, 0)),
            pl.BlockSpec((1, 1), lambda i: (0, 0)),
        ],
        out_specs=pl.BlockSpec((NUM_GRAPHS, 1), lambda i: (0, 0)),
        out_shape=jax.ShapeDtypeStruct((NUM_GRAPHS, 1), jnp.float32),
        scratch_shapes=[
            pltpu.VMEM((NUM_GRAPHS, D_FEAT), jnp.float32),
            pltpu.VMEM((16, NUM_GRAPHS), jnp.float32),
        ],
    )(partials, counts,
      W1, b1.reshape(1, HIDDEN), W2, b2.reshape(1, HIDDEN),
      W3, b3.reshape(1, 1))
    return pred.reshape(NUM_GRAPHS)


# speedup vs baseline: 1.2986x; 1.0279x over previous
import functools

import jax
import jax.numpy as jnp
from jax import lax
from jax.experimental import pallas as pl
from jax.experimental.pallas import tpu as pltpu
from jax.experimental.pallas import tpu_sc as plsc

N_NODES = 100000
D_FEAT = 128
NUM_GRAPHS = 512
HIDDEN = 256

NC = 2
NS = 16
NW = NC * NS
ROWS_W = 3136
CHUNK = 208
NPAIRS = 8
NGROUPS = CHUNK // 16
NSLICE = D_FEAT // 16
TCBLK = 8

_GDN = lax.GatherDimensionNumbers(
    offset_dims=(), collapsed_slice_dims=(0,), start_index_map=(0,))


def _lane_bcast(v, r):
    idx = jnp.full((16,), r, dtype=jnp.int32)
    return lax.gather(v, idx[:, None], _GDN, (1,),
                      mode=lax.GatherScatterMode.PROMISE_IN_BOUNDS)


def _sc_body(feat_hbm, ids_hbm, out_hbm, cnt_hbm,
             rowA, idsA, rowB, idsB, acc, cnt, semA, semB):
    cid = lax.axis_index("c")
    sid = lax.axis_index("s")
    wid = sid * NC + cid
    base = wid * ROWS_W
    end = jnp.minimum(base + ROWS_W, N_NODES)

    zero16 = jnp.zeros((16,), jnp.float32)
    ones16 = jnp.ones((16,), jnp.float32)
    iota16 = lax.iota(jnp.int32, 16)

    def start_chunk(k, rowbuf, idsbuf, sem):
        b = jnp.minimum(base + k * CHUNK, end - CHUNK)
        pltpu.async_copy(feat_hbm.at[pl.ds(b, CHUNK), :], rowbuf, sem)
        pltpu.async_copy(ids_hbm.at[pl.ds(b, CHUNK)], idsbuf, sem)

    def wait_chunk(rowbuf, idsbuf, sem):
        pltpu.make_async_copy(
            feat_hbm.at[pl.ds(0, CHUNK), :], rowbuf, sem).wait()
        pltpu.make_async_copy(
            ids_hbm.at[pl.ds(0, CHUNK)], idsbuf, sem).wait()

    start_chunk(0, rowA, idsA, semA)

    def zrow(i, carry):
        for j in range(NSLICE):
            acc[i, pl.ds(j * 16, 16)] = zero16
        return carry

    lax.fori_loop(0, NUM_GRAPHS, zrow, 0)

    def zcnt(c, carry):
        for r in range(16):
            cnt[r, pl.ds(c * 16, 16)] = zero16
        return carry

    lax.fori_loop(0, NUM_GRAPHS // 16, zcnt, 0)

    def make_group_body(rowbuf, idsbuf):
        def group_body(g, carry):
            accs, segv = carry
            idv = idsbuf[pl.ds(g * 16, 16)]
            plsc.addupdate_scatter(cnt, [iota16, idv], ones16)
            nb = plsc.all_reduce_population_count(idv != segv)[0]

            def fast(ops):
                a, sv = ops
                new = list(a)
                for r in range(16):
                    row = g * 16 + r
                    for j in range(NSLICE):
                        new[j] = new[j] + rowbuf[row, pl.ds(j * 16, 16)]
                return tuple(new), sv

            def slow(ops):
                a, sv = ops
                tgt = jnp.maximum(sv, 0)
                for j in range(NSLICE):
                    plsc.addupdate_scatter(acc, [tgt, iota16 + j * 16], a[j])
                prev = None
                for r in range(16):
                    seg = _lane_bcast(idv, r)
                    row = g * 16 + r
                    xs = [rowbuf[row, pl.ds(j * 16, 16)]
                          for j in range(NSLICE)]
                    if prev is not None:
                        pseg, pxs = prev
                        for j in range(NSLICE):
                            plsc.addupdate_scatter(
                                acc, [pseg, iota16 + j * 16], pxs[j])
                    prev = (seg, xs)
                pseg, pxs = prev
                for j in range(NSLICE):
                    plsc.addupdate_scatter(acc, [pseg, iota16 + j * 16],
                                           pxs[j])
                zeros = tuple(zero16 for _ in range(NSLICE))
                return zeros, _lane_bcast(idv, 15)

            return lax.cond(nb == 0, fast, slow, (accs, segv))
        return group_body

    def process(k, state, rowbuf, idsbuf):
        p, accs, segv = state
        b = jnp.minimum(base + k * CHUNK, end - CHUNK)
        gs = (p - b) // 16
        accs, segv = lax.fori_loop(
            gs, NGROUPS, make_group_body(rowbuf, idsbuf), (accs, segv))
        return b + CHUNK, accs, segv

    def pair_body(t, state):
        k0 = 2 * t
        start_chunk(k0 + 1, rowB, idsB, semB)
        wait_chunk(rowA, idsA, semA)
        state = process(k0, state, rowA, idsA)

        @pl.when(t < NPAIRS - 1)
        def _():
            start_chunk(k0 + 2, rowA, idsA, semA)

        wait_chunk(rowB, idsB, semB)
        return process(k0 + 1, state, rowB, idsB)

    init_state = (base,
                  tuple(zero16 for _ in range(NSLICE)),
                  jnp.full((16,), -1, jnp.int32))
    _, accs, segv = lax.fori_loop(0, NPAIRS, pair_body, init_state)

    tgt = jnp.maximum(segv, 0)
    for j in range(NSLICE):
        plsc.addupdate_scatter(acc, [tgt, iota16 + j * 16], accs[j])

    pltpu.sync_copy(acc, out_hbm.at[pl.ds(wid * NUM_GRAPHS, NUM_GRAPHS), :])
    pltpu.sync_copy(cnt, cnt_hbm.at[pl.ds(wid * 16, 16), :])


_sc_seg_sum = functools.partial(
    pl.kernel,
    out_type=[
        jax.ShapeDtypeStruct((NW * NUM_GRAPHS, D_FEAT), jnp.float32),
        jax.ShapeDtypeStruct((NW * 16, NUM_GRAPHS), jnp.float32),
    ],
    mesh=plsc.VectorSubcoreMesh(
        core_axis_name="c", subcore_axis_name="s",
        num_cores=NC, num_subcores=NS),
    compiler_params=pltpu.CompilerParams(
        needs_layout_passes=False, use_tc_tiling_on_sc=False),
    scratch_types=[
        pltpu.VMEM((CHUNK, D_FEAT), jnp.float32),
        pltpu.VMEM((CHUNK,), jnp.int32),
        pltpu.VMEM((CHUNK, D_FEAT), jnp.float32),
        pltpu.VMEM((CHUNK,), jnp.int32),
        pltpu.VMEM((NUM_GRAPHS, D_FEAT), jnp.float32),
        pltpu.VMEM((16, NUM_GRAPHS), jnp.float32),
        pltpu.SemaphoreType.DMA,
        pltpu.SemaphoreType.DMA,
    ],
)(_sc_body)


def _tc_body(p_ref, c_ref, W1_ref, b1_ref, W2_ref, b2_ref, W3_ref, b3_ref,
             out_ref, acc_ref, cacc_ref):
    i = pl.program_id(0)
    psum = p_ref[0:NUM_GRAPHS]
    csum = c_ref[0:16]
    for w in range(1, TCBLK):
        psum = psum + p_ref[w * NUM_GRAPHS:(w + 1) * NUM_GRAPHS]
        csum = csum + c_ref[w * 16:(w + 1) * 16]

    @pl.when(i == 0)
    def _init():
        acc_ref[...] = psum
        cacc_ref[...] = csum

    @pl.when(i > 0)
    def _accum():
        acc_ref[...] += psum
        cacc_ref[...] += csum

    @pl.when(i == NW // TCBLK - 1)
    def _finish():
        c_col = jax.lax.dot_general(
            cacc_ref[...], jnp.ones((16, 1), jnp.float32),
            (((0,), (0,)), ((), ())),
            preferred_element_type=jnp.float32)
        pooled = acc_ref[...] / jnp.maximum(c_col, 1.0)
        h = jnp.maximum(
            jnp.dot(pooled, W1_ref[...], preferred_element_type=jnp.float32)
            + b1_ref[...], 0.0)
        h = jnp.maximum(
            jnp.dot(h, W2_ref[...], preferred_element_type=jnp.float32)
            + b2_ref[...], 0.0)
        out_ref[...] = (
            jnp.dot(h, W3_ref[...], preferred_element_type=jnp.float32)
            + b3_ref[...])


def kernel(feat, segment_ids, W1, b1, W2, b2, W3, b3):
    ids = segment_ids.astype(jnp.int32)
    partials, counts = _sc_seg_sum(feat, ids)
    pred = pl.pallas_call(
        _tc_body,
        grid=(NW // TCBLK,),
        in_specs=[
            pl.BlockSpec((TCBLK * NUM_GRAPHS, D_FEAT), lambda i: (i, 0)),
            pl.BlockSpec((TCBLK * 16, NUM_GRAPHS), lambda i: (i, 0)),
            pl.BlockSpec((D_FEAT, HIDDEN), lambda i: (0, 0)),
            pl.BlockSpec((1, HIDDEN), lambda i: (0, 0)),
            pl.BlockSpec((HIDDEN, HIDDEN), lambda i: (0, 0)),
            pl.BlockSpec((1, HIDDEN), lambda i: (0, 0)),
            pl.BlockSpec((HIDDEN, 1), lambda i: (0, 0)),
            pl.BlockSpec((1, 1), lambda i: (0, 0)),
        ],
        out_specs=pl.BlockSpec((NUM_GRAPHS, 1), lambda i: (0, 0)),
        out_shape=jax.ShapeDtypeStruct((NUM_GRAPHS, 1), jnp.float32),
        scratch_shapes=[
            pltpu.VMEM((NUM_GRAPHS, D_FEAT), jnp.float32),
            pltpu.VMEM((16, NUM_GRAPHS), jnp.float32),
        ],
    )(partials, counts,
      W1, b1.reshape(1, HIDDEN), W2, b2.reshape(1, HIDDEN),
      W3, b3.reshape(1, 1))
    return pred.reshape(NUM_GRAPHS)


# speedup vs baseline: 1.3080x; 1.0072x over previous
import functools

import jax
import jax.numpy as jnp
from jax import lax
from jax.experimental import pallas as pl
from jax.experimental.pallas import tpu as pltpu
from jax.experimental.pallas import tpu_sc as plsc

N_NODES = 100000
D_FEAT = 128
NUM_GRAPHS = 512
HIDDEN = 256

NC = 2
NS = 16
NW = NC * NS
ROWS_W = 3136
CHUNK = 208
NPAIRS = 8
NGROUPS = CHUNK // 16
NSLICE = D_FEAT // 16
TCBLK = 16

_GDN = lax.GatherDimensionNumbers(
    offset_dims=(), collapsed_slice_dims=(0,), start_index_map=(0,))


def _lane_bcast(v, r):
    idx = jnp.full((16,), r, dtype=jnp.int32)
    return lax.gather(v, idx[:, None], _GDN, (1,),
                      mode=lax.GatherScatterMode.PROMISE_IN_BOUNDS)


def _sc_body(feat_hbm, ids_hbm, out_hbm, cnt_hbm,
             rowA, idsA, rowB, idsB, acc, cnt, semA, semB):
    cid = lax.axis_index("c")
    sid = lax.axis_index("s")
    wid = sid * NC + cid
    base = wid * ROWS_W
    end = jnp.minimum(base + ROWS_W, N_NODES)

    zero16 = jnp.zeros((16,), jnp.float32)
    ones16 = jnp.ones((16,), jnp.float32)
    iota16 = lax.iota(jnp.int32, 16)

    def start_chunk(k, rowbuf, idsbuf, sem):
        b = jnp.minimum(base + k * CHUNK, end - CHUNK)
        pltpu.async_copy(feat_hbm.at[pl.ds(b, CHUNK), :], rowbuf, sem)
        pltpu.async_copy(ids_hbm.at[pl.ds(b, CHUNK)], idsbuf, sem)

    def wait_chunk(rowbuf, idsbuf, sem):
        pltpu.make_async_copy(
            feat_hbm.at[pl.ds(0, CHUNK), :], rowbuf, sem).wait()
        pltpu.make_async_copy(
            ids_hbm.at[pl.ds(0, CHUNK)], idsbuf, sem).wait()

    start_chunk(0, rowA, idsA, semA)

    def zrow(i, carry):
        for j in range(NSLICE):
            acc[i, pl.ds(j * 16, 16)] = zero16
        return carry

    lax.fori_loop(0, NUM_GRAPHS, zrow, 0)

    def zcnt(c, carry):
        for r in range(16):
            cnt[r, pl.ds(c * 16, 16)] = zero16
        return carry

    lax.fori_loop(0, NUM_GRAPHS // 16, zcnt, 0)

    def make_group_body(rowbuf, idsbuf):
        def group_body(g, carry):
            accs, segv = carry
            idv = idsbuf[pl.ds(g * 16, 16)]
            plsc.addupdate_scatter(cnt, [iota16, idv], ones16)
            nb = plsc.all_reduce_population_count(idv != segv)[0]

            def fast(ops):
                a, sv = ops
                new = list(a)
                for r in range(16):
                    row = g * 16 + r
                    for j in range(NSLICE):
                        new[j] = new[j] + rowbuf[row, pl.ds(j * 16, 16)]
                return tuple(new), sv

            def slow(ops):
                a, sv = ops
                tgt = jnp.maximum(sv, 0)
                for j in range(NSLICE):
                    plsc.addupdate_scatter(acc, [tgt, iota16 + j * 16], a[j])
                prev = None
                for r in range(16):
                    seg = _lane_bcast(idv, r)
                    row = g * 16 + r
                    xs = [rowbuf[row, pl.ds(j * 16, 16)]
                          for j in range(NSLICE)]
                    if prev is not None:
                        pseg, pxs = prev
                        for j in range(NSLICE):
                            plsc.addupdate_scatter(
                                acc, [pseg, iota16 + j * 16], pxs[j])
                    prev = (seg, xs)
                pseg, pxs = prev
                for j in range(NSLICE):
                    plsc.addupdate_scatter(acc, [pseg, iota16 + j * 16],
                                           pxs[j])
                zeros = tuple(zero16 for _ in range(NSLICE))
                return zeros, _lane_bcast(idv, 15)

            return lax.cond(nb == 0, fast, slow, (accs, segv))
        return group_body

    def process(k, state, rowbuf, idsbuf):
        p, accs, segv = state
        b = jnp.minimum(base + k * CHUNK, end - CHUNK)
        gs = (p - b) // 16
        accs, segv = lax.fori_loop(
            gs, NGROUPS, make_group_body(rowbuf, idsbuf), (accs, segv))
        return b + CHUNK, accs, segv

    def pair_body(t, state):
        k0 = 2 * t
        start_chunk(k0 + 1, rowB, idsB, semB)
        wait_chunk(rowA, idsA, semA)
        state = process(k0, state, rowA, idsA)

        @pl.when(t < NPAIRS - 1)
        def _():
            start_chunk(k0 + 2, rowA, idsA, semA)

        wait_chunk(rowB, idsB, semB)
        return process(k0 + 1, state, rowB, idsB)

    init_state = (base,
                  tuple(zero16 for _ in range(NSLICE)),
                  jnp.full((16,), -1, jnp.int32))
    _, accs, segv = lax.fori_loop(0, NPAIRS, pair_body, init_state)

    tgt = jnp.maximum(segv, 0)
    for j in range(NSLICE):
        plsc.addupdate_scatter(acc, [tgt, iota16 + j * 16], accs[j])

    pltpu.sync_copy(acc, out_hbm.at[pl.ds(wid * NUM_GRAPHS, NUM_GRAPHS), :])
    pltpu.sync_copy(cnt, cnt_hbm.at[pl.ds(wid * 16, 16), :])


_sc_seg_sum = functools.partial(
    pl.kernel,
    out_type=[
        jax.ShapeDtypeStruct((NW * NUM_GRAPHS, D_FEAT), jnp.float32),
        jax.ShapeDtypeStruct((NW * 16, NUM_GRAPHS), jnp.float32),
    ],
    mesh=plsc.VectorSubcoreMesh(
        core_axis_name="c", subcore_axis_name="s",
        num_cores=NC, num_subcores=NS),
    compiler_params=pltpu.CompilerParams(
        needs_layout_passes=False, use_tc_tiling_on_sc=False),
    scratch_types=[
        pltpu.VMEM((CHUNK, D_FEAT), jnp.float32),
        pltpu.VMEM((CHUNK,), jnp.int32),
        pltpu.VMEM((CHUNK, D_FEAT), jnp.float32),
        pltpu.VMEM((CHUNK,), jnp.int32),
        pltpu.VMEM((NUM_GRAPHS, D_FEAT), jnp.float32),
        pltpu.VMEM((16, NUM_GRAPHS), jnp.float32),
        pltpu.SemaphoreType.DMA,
        pltpu.SemaphoreType.DMA,
    ],
)(_sc_body)


def _tc_body(p_ref, c_ref, W1_ref, b1_ref, W2_ref, b2_ref, W3_ref, b3_ref,
             out_ref, acc_ref, cacc_ref):
    i = pl.program_id(0)
    psum = p_ref[0:NUM_GRAPHS]
    csum = c_ref[0:16]
    for w in range(1, TCBLK):
        psum = psum + p_ref[w * NUM_GRAPHS:(w + 1) * NUM_GRAPHS]
        csum = csum + c_ref[w * 16:(w + 1) * 16]

    @pl.when(i == 0)
    def _init():
        acc_ref[...] = psum
        cacc_ref[...] = csum

    @pl.when(i > 0)
    def _accum():
        acc_ref[...] += psum
        cacc_ref[...] += csum

    @pl.when(i == NW // TCBLK - 1)
    def _finish():
        c_col = jax.lax.dot_general(
            cacc_ref[...], jnp.ones((16, 1), jnp.float32),
            (((0,), (0,)), ((), ())),
            preferred_element_type=jnp.float32)
        pooled = acc_ref[...] / jnp.maximum(c_col, 1.0)
        h = jnp.maximum(
            jnp.dot(pooled, W1_ref[...], preferred_element_type=jnp.float32)
            + b1_ref[...], 0.0)
        h = jnp.maximum(
            jnp.dot(h, W2_ref[...], preferred_element_type=jnp.float32)
            + b2_ref[...], 0.0)
        out_ref[...] = (
            jnp.dot(h, W3_ref[...], preferred_element_type=jnp.float32)
            + b3_ref[...])


def kernel(feat, segment_ids, W1, b1, W2, b2, W3, b3):
    ids = segment_ids.astype(jnp.int32)
    partials, counts = _sc_seg_sum(feat, ids)
    pred = pl.pallas_call(
        _tc_body,
        grid=(NW // TCBLK,),
        in_specs=[
            pl.BlockSpec((TCBLK * NUM_GRAPHS, D_FEAT), lambda i: (i, 0)),
            pl.BlockSpec((TCBLK * 16, NUM_GRAPHS), lambda i: (i, 0)),
            pl.BlockSpec((D_FEAT, HIDDEN), lambda i: (0, 0)),
            pl.BlockSpec((1, HIDDEN), lambda i: (0, 0)),
            pl.BlockSpec((HIDDEN, HIDDEN), lambda i: (0, 0)),
            pl.BlockSpec((1, HIDDEN), lambda i: (0, 0)),
            pl.BlockSpec((HIDDEN, 1), lambda i: (0, 0)),
            pl.BlockSpec((1, 1), lambda i: (0, 0)),
        ],
        out_specs=pl.BlockSpec((NUM_GRAPHS, 1), lambda i: (0, 0)),
        out_shape=jax.ShapeDtypeStruct((NUM_GRAPHS, 1), jnp.float32),
        scratch_shapes=[
            pltpu.VMEM((NUM_GRAPHS, D_FEAT), jnp.float32),
            pltpu.VMEM((16, NUM_GRAPHS), jnp.float32),
        ],
    )(partials, counts,
      W1, b1.reshape(1, HIDDEN), W2, b2.reshape(1, HIDDEN),
      W3, b3.reshape(1, 1))
    return pred.reshape(NUM_GRAPHS)
